# baseline (speedup 1.0000x reference)
import jax
import jax.numpy as jnp
from jax.experimental import pallas as pl
from jax.experimental.pallas import tpu as pltpu

_R_BLK = 8


def _copy_kernel(scores_ref, out_ref):
    out_ref[...] = scores_ref[...]


def kernel(input_ids, scores):
    batch, vocab = scores.shape
    return pl.pallas_call(
        _copy_kernel,
        grid=(batch // _R_BLK,),
        in_specs=[pl.BlockSpec((_R_BLK, vocab), lambda i: (i, 0))],
        out_specs=pl.BlockSpec((_R_BLK, vocab), lambda i: (i, 0)),
        out_shape=jax.ShapeDtypeStruct(scores.shape, scores.dtype),
        compiler_params=pltpu.CompilerParams(
            dimension_semantics=("parallel",)),
    )(scores)
